# Initial kernel scaffold; baseline (speedup 1.0000x reference)
import functools
import math

import jax
import jax.numpy as jnp
import numpy as np
from jax import lax
from jax.experimental import pallas as pl
from jax.experimental.pallas import tpu as pltpu
from jax.experimental.pallas import tpu_sc as plsc

_FIELD_DIM = 100000
_NF = 26
_D = 16
_B = 16384
_BF = _B * _NF
_CONCAT = _NF * _D
_EPS_C = 1.0 / math.sqrt(1.0 + 1e-5)

_NC = 2
_NS = 16
_L = 16
_NW = _NC * _NS
_PER_W = _BF // _NW
_CHUNK = 1664
_ROWS = _CHUNK // 128
_NCHUNK = _PER_W // _CHUNK

_OFF_NP = ((np.arange(_CHUNK) % _NF) * _FIELD_DIM).astype(np.int32).reshape(
    _ROWS, 128)


def _sc_gather_body(x_hbm, off_hbm, embt_hbm, lint_hbm,
                    emb_out, lin_out,
                    idx_v, off_v, xo_v, emb_buf, lin_buf, sem_e, sem_l):
    wid = lax.axis_index("s") * _NC + lax.axis_index("c")
    row_base = wid * (_PER_W // 128)
    pltpu.sync_copy(off_hbm, off_v)

    def chunk(c, carry):
        r0 = row_base + c * _ROWS
        pltpu.sync_copy(x_hbm.at[pl.ds(r0, _ROWS)], idx_v)
        for r in range(_ROWS):
            for i in range(128 // _L):
                s = pl.ds(i * _L, _L)
                xo_v[r, s] = idx_v[r, s] + off_v[r, s]
        cps = []
        for r in range(_ROWS):
            idx_r = xo_v.at[r]
            cps.append(pltpu.async_copy(
                embt_hbm.at[idx_r], emb_buf.at[pl.ds(r * 128, 128)], sem_e))
            cps.append(pltpu.async_copy(
                lint_hbm.at[idx_r], lin_buf.at[r], sem_l))
        for cp in cps:
            cp.wait()
        pltpu.sync_copy(emb_buf, emb_out.at[pl.ds(r0 * 128, _CHUNK)])
        pltpu.sync_copy(lin_buf, lin_out.at[pl.ds(r0, _ROWS)])
        return carry

    lax.fori_loop(0, _NCHUNK, chunk, 0)


_sc_gather = functools.partial(
    pl.kernel,
    mesh=plsc.VectorSubcoreMesh(core_axis_name="c", subcore_axis_name="s"),
    out_type=[
        jax.ShapeDtypeStruct((_BF, _D), jnp.float32),
        jax.ShapeDtypeStruct((_BF // 128, 128), jnp.float32),
    ],
    scratch_types=[
        pltpu.VMEM((_ROWS, 128), jnp.int32),
        pltpu.VMEM((_ROWS, 128), jnp.int32),
        pltpu.VMEM((_ROWS, 128), jnp.int32),
        pltpu.VMEM((_CHUNK, _D), jnp.float32),
        pltpu.VMEM((_ROWS, 128), jnp.float32),
        pltpu.SemaphoreType.DMA,
        pltpu.SemaphoreType.DMA,
    ],
)(_sc_gather_body)


def _mlp_body(emb_ref, lin_ref, w1_ref, g1_ref, b1_ref, be1_ref,
              w2_ref, g2_ref, b2_ref, be2_ref, w3_ref, sc_ref, out_ref):
    emb = emb_ref[...]
    h = jnp.dot(emb, w1_ref[...], preferred_element_type=jnp.float32)
    a1 = g1_ref[...] * _EPS_C
    h = jnp.maximum(h * a1 + (b1_ref[...] * a1 + be1_ref[...]), 0.0)
    h = jnp.dot(h, w2_ref[...], preferred_element_type=jnp.float32)
    a2 = g2_ref[...] * _EPS_C
    h = jnp.maximum(h * a2 + (b2_ref[...] * a2 + be2_ref[...]), 0.0)
    y = jnp.sum(h * w3_ref[...], axis=1, keepdims=True)
    y = y + jnp.sum(lin_ref[...], axis=1, keepdims=True) + sc_ref[0, 0]
    out_ref[...] = 1.0 / (1.0 + jnp.exp(-y))


_BB = 2048


def _mlp_call(emb, lin, w1, g1, b1, be1, w2, g2, b2, be2, w3, sc):
    grid = (_B // _BB,)
    full = lambda shape: pl.BlockSpec(shape, lambda i: (0, 0))
    return pl.pallas_call(
        _mlp_body,
        grid=grid,
        in_specs=[
            pl.BlockSpec((_BB, _CONCAT), lambda i: (i, 0)),
            pl.BlockSpec((_BB, _NF), lambda i: (i, 0)),
            full((_CONCAT, 64)), full((1, 64)), full((1, 64)), full((1, 64)),
            full((64, 32)), full((1, 32)), full((1, 32)), full((1, 32)),
            full((1, 32)), full((1, 1)),
        ],
        out_specs=pl.BlockSpec((_BB, 1), lambda i: (i, 0)),
        out_shape=jax.ShapeDtypeStruct((_B, 1), jnp.float32),
    )(emb, lin, w1, g1, b1, be1, w2, g2, b2, be2, w3, sc)


def kernel(x, embed_table, linear_table, bias, W1, b1, g1, be1,
           W2, b2, g2, be2, W3, b3):
    x2 = x.reshape(_BF // 128, 128)
    off2 = jnp.asarray(_OFF_NP)
    emb_flat, lin_flat = _sc_gather(
        x2, off2, embed_table, linear_table.reshape(-1))
    emb = emb_flat.reshape(_B, _CONCAT)
    lin = lin_flat.reshape(_B, _NF)
    sc = (bias + b3).reshape(1, 1)
    out = _mlp_call(
        emb, lin, W1,
        g1.reshape(1, -1), b1.reshape(1, -1), be1.reshape(1, -1),
        W2, g2.reshape(1, -1), b2.reshape(1, -1), be2.reshape(1, -1),
        W3.reshape(1, -1), sc)
    return out.reshape(_B)


# speedup vs baseline: 6.3693x; 6.3693x over previous
import functools
import math

import jax
import jax.numpy as jnp
import numpy as np
from jax import lax
from jax.experimental import pallas as pl
from jax.experimental.pallas import tpu as pltpu
from jax.experimental.pallas import tpu_sc as plsc

_FIELD_DIM = 100000
_NF = 26
_D = 16
_B = 16384
_BF = _B * _NF
_CONCAT = _NF * _D
_EPS_C = 1.0 / math.sqrt(1.0 + 1e-5)

_NC = 2
_NS = 16
_L = 16
_NW = _NC * _NS
_PER_W = _BF // _NW
_WROWS = _PER_W // 128
_CHUNK = 1024
_ROWS = _CHUNK // 128
_NCHUNK = _PER_W // _CHUNK

_OFF_NP = ((np.arange(_PER_W) % _NF) * _FIELD_DIM).astype(np.int32).reshape(
    _WROWS, 128)


def _sc_gather_body(x_hbm, off_hbm, embt_hbm, lint_hbm,
                    emb_out, lin_out,
                    idx_v, off_v, xo_v, emb_buf, lin_buf, sem_e, sem_l):
    wid = lax.axis_index("s") * _NC + lax.axis_index("c")
    row_base = pl.multiple_of(wid * _WROWS, 8)
    pltpu.sync_copy(off_hbm, off_v)

    for c in range(_NCHUNK):
        r0 = row_base + c * _ROWS
        pltpu.sync_copy(x_hbm.at[pl.ds(r0, _ROWS)], idx_v)
        for r in range(_ROWS):
            for i in range(128 // _L):
                s = pl.ds(i * _L, _L)
                xo_v[r, s] = idx_v[r, s] + off_v[c * _ROWS + r, s]
        cps = []
        for r in range(_ROWS):
            idx_r = xo_v.at[r]
            cps.append(pltpu.async_copy(
                embt_hbm.at[idx_r], emb_buf.at[pl.ds(r * 128, 128)], sem_e))
            cps.append(pltpu.async_copy(
                lint_hbm.at[idx_r], lin_buf.at[r], sem_l))
        for cp in cps:
            cp.wait()
        pltpu.sync_copy(emb_buf, emb_out.at[pl.ds(r0 * 128, _CHUNK)])
        pltpu.sync_copy(lin_buf, lin_out.at[pl.ds(r0, _ROWS)])


_sc_gather = functools.partial(
    pl.kernel,
    mesh=plsc.VectorSubcoreMesh(core_axis_name="c", subcore_axis_name="s"),
    compiler_params=pltpu.CompilerParams(use_tc_tiling_on_sc=False),
    out_type=[
        jax.ShapeDtypeStruct((_BF, _D), jnp.float32),
        jax.ShapeDtypeStruct((_BF // 128, 128), jnp.float32),
    ],
    scratch_types=[
        pltpu.VMEM((_ROWS, 128), jnp.int32),
        pltpu.VMEM((_WROWS, 128), jnp.int32),
        pltpu.VMEM((_ROWS, 128), jnp.int32),
        pltpu.VMEM((_CHUNK, _D), jnp.float32),
        pltpu.VMEM((_ROWS, 128), jnp.float32),
        pltpu.SemaphoreType.DMA,
        pltpu.SemaphoreType.DMA,
    ],
)(_sc_gather_body)


def _mlp_body(emb_ref, lin_ref, w1_ref, g1_ref, b1_ref, be1_ref,
              w2_ref, g2_ref, b2_ref, be2_ref, w3_ref, sc_ref, out_ref):
    emb = emb_ref[...]
    h = jnp.dot(emb, w1_ref[...], preferred_element_type=jnp.float32)
    a1 = g1_ref[...] * _EPS_C
    h = jnp.maximum(h * a1 + (b1_ref[...] * a1 + be1_ref[...]), 0.0)
    h = jnp.dot(h, w2_ref[...], preferred_element_type=jnp.float32)
    a2 = g2_ref[...] * _EPS_C
    h = jnp.maximum(h * a2 + (b2_ref[...] * a2 + be2_ref[...]), 0.0)
    y = jnp.sum(h * w3_ref[...], axis=1, keepdims=True)
    y = y + jnp.sum(lin_ref[...], axis=1, keepdims=True) + sc_ref[0, 0]
    out_ref[...] = 1.0 / (1.0 + jnp.exp(-y))


_BB = 2048


def _mlp_call(emb, lin, w1, g1, b1, be1, w2, g2, b2, be2, w3, sc):
    grid = (_B // _BB,)
    full = lambda shape: pl.BlockSpec(shape, lambda i: (0, 0))
    return pl.pallas_call(
        _mlp_body,
        grid=grid,
        in_specs=[
            pl.BlockSpec((_BB, _CONCAT), lambda i: (i, 0)),
            pl.BlockSpec((_BB, _NF), lambda i: (i, 0)),
            full((_CONCAT, 64)), full((1, 64)), full((1, 64)), full((1, 64)),
            full((64, 32)), full((1, 32)), full((1, 32)), full((1, 32)),
            full((1, 32)), full((1, 1)),
        ],
        out_specs=pl.BlockSpec((_BB, 1), lambda i: (i, 0)),
        out_shape=jax.ShapeDtypeStruct((_B, 1), jnp.float32),
    )(emb, lin, w1, g1, b1, be1, w2, g2, b2, be2, w3, sc)


def kernel(x, embed_table, linear_table, bias, W1, b1, g1, be1,
           W2, b2, g2, be2, W3, b3):
    x2 = x.reshape(_BF // 128, 128)
    off2 = jnp.asarray(_OFF_NP)
    emb_flat, lin_flat = _sc_gather(
        x2, off2, embed_table, linear_table.reshape(-1))
    emb = emb_flat.reshape(_B, _CONCAT)
    lin = lin_flat.reshape(_B, _NF)
    sc = (bias + b3).reshape(1, 1)
    out = _mlp_call(
        emb, lin, W1,
        g1.reshape(1, -1), b1.reshape(1, -1), be1.reshape(1, -1),
        W2, g2.reshape(1, -1), b2.reshape(1, -1), be2.reshape(1, -1),
        W3.reshape(1, -1), sc)
    return out.reshape(_B)
